# baseline (speedup 1.0000x reference)
import jax
import jax.numpy as jnp
from jax.experimental import pallas as pl
from jax.experimental.pallas import tpu as pltpu

P = 128
P_REAL = 100
N_GRAPHS = 100
D_H = 128
NUM_BASES = 64
RADIUS = 4.0
I_CHUNK = 32


def _cfconv_kernel(x_ref, r_ref, rt_ref, mu_ref, w1_ref, w2_ref, out_ref):
    Xg = x_ref[0]
    Rg = r_ref[0]
    Rt = rt_ref[0]
    mu = mu_ref[0]
    delta = mu[1] - mu[0]
    gamma = 1.0 / (2.0 * delta * delta)

    r2c = jnp.sum(Rg * Rg, axis=1)[:, None]
    r2r = jnp.sum(Rt * Rt, axis=0, keepdims=True)
    G = jax.lax.dot_general(Rg, Rg, (((1,), (1,)), ((), ())),
                            preferred_element_type=jnp.float32)
    D_mask = (r2c + r2r) - 2.0 * G
    Dd = ((Rg[:, 0:1] - Rt[0:1, :]) ** 2
          + (Rg[:, 1:2] - Rt[1:2, :]) ** 2
          + (Rg[:, 2:3] - Rt[2:3, :]) ** 2)

    ii = jax.lax.broadcasted_iota(jnp.int32, (P, P), 0)
    jj = jax.lax.broadcasted_iota(jnp.int32, (P, P), 1)
    mask = ((D_mask <= RADIUS * RADIUS) & (ii != jj)
            & (ii < P_REAL) & (jj < P_REAL))
    D = jnp.where(mask, Dd, 1e4)

    acc = jnp.zeros((P, D_H), dtype=jnp.float32)
    for c in range(P // I_CHUNK):
        sl = slice(c * I_CHUNK, (c + 1) * I_CHUNK)
        Dc = D[sl, :]
        phi = jnp.exp(-gamma * (Dc[:, :, None] - mu[None, None, :]) ** 2
                      ).astype(jnp.bfloat16)
        phif = phi.reshape(I_CHUNK * P, NUM_BASES)
        h = jnp.maximum(
            jnp.dot(phif, w1_ref[...].astype(jnp.bfloat16),
                    preferred_element_type=jnp.float32), 0.0).astype(jnp.bfloat16)
        m = jnp.maximum(
            jnp.dot(h, w2_ref[...].astype(jnp.bfloat16),
                    preferred_element_type=jnp.float32), 0.0)
        m3 = m.reshape(I_CHUNK, P, D_H)
        contrib = Xg[sl, None, :] * m3
        acc = acc + jnp.sum(contrib, axis=0)
    out_ref[0] = acc


@jax.jit
def kernel(X, R, batch_index, mu, W1, W2):
    del batch_index
    n = X.shape[0]
    R3 = jnp.pad(R.reshape(N_GRAPHS, P_REAL, R.shape[1]),
                 ((0, 0), (0, P - P_REAL), (0, 8 - R.shape[1])))
    Rt3 = jnp.transpose(R3, (0, 2, 1))
    Xp = jnp.pad(X.reshape(N_GRAPHS, P_REAL, D_H),
                 ((0, 0), (0, P - P_REAL), (0, 0)))
    mu2 = mu.reshape(1, NUM_BASES)

    Hp = pl.pallas_call(
        _cfconv_kernel,
        grid=(N_GRAPHS,),
        in_specs=[
            pl.BlockSpec((1, P, D_H), lambda g: (g, 0, 0)),
            pl.BlockSpec((1, P, 8), lambda g: (g, 0, 0)),
            pl.BlockSpec((1, 8, P), lambda g: (g, 0, 0)),
            pl.BlockSpec((1, NUM_BASES), lambda g: (0, 0)),
            pl.BlockSpec((NUM_BASES, D_H), lambda g: (0, 0)),
            pl.BlockSpec((D_H, D_H), lambda g: (0, 0)),
        ],
        out_specs=pl.BlockSpec((1, P, D_H), lambda g: (g, 0, 0)),
        out_shape=jax.ShapeDtypeStruct((N_GRAPHS, P, D_H), jnp.float32),
        compiler_params=pltpu.CompilerParams(
            dimension_semantics=("parallel",)),
    )(Xp, R3, Rt3, mu2, W1, W2)

    return Hp[:, :P_REAL, :].reshape(n, D_H)


# speedup vs baseline: 1.1730x; 1.1730x over previous
import jax
import jax.numpy as jnp
from jax.experimental import pallas as pl
from jax.experimental.pallas import tpu as pltpu

P = 128
P_REAL = 100
N_GRAPHS = 100
D_H = 128
NUM_BASES = 64
RADIUS = 4.0
I_CHUNK = 32


def _cfconv_kernel(x_ref, r_ref, rt_ref, mu_ref, w1_ref, w2_ref, out_ref):
    Xg = x_ref[0]
    Rg = r_ref[0]
    Rt = rt_ref[0]
    mu = mu_ref[0]
    delta = mu[1] - mu[0]
    gamma = 1.0 / (2.0 * delta * delta)

    r2c = jnp.sum(Rg * Rg, axis=1)[:, None]
    r2r = jnp.sum(Rt * Rt, axis=0, keepdims=True)
    G = jax.lax.dot_general(Rg, Rg, (((1,), (1,)), ((), ())),
                            preferred_element_type=jnp.float32)
    D_mask = (r2c + r2r) - 2.0 * G
    Dd = ((Rg[:, 0:1] - Rt[0:1, :]) ** 2
          + (Rg[:, 1:2] - Rt[1:2, :]) ** 2
          + (Rg[:, 2:3] - Rt[2:3, :]) ** 2)

    ii = jax.lax.broadcasted_iota(jnp.int32, (P, P), 0)
    jj = jax.lax.broadcasted_iota(jnp.int32, (P, P), 1)
    mask = ((D_mask <= RADIUS * RADIUS) & (ii != jj)
            & (ii < P_REAL) & (jj < P_REAL))
    D = jnp.where(mask, Dd, 1e4)

    acc = jnp.zeros((P, D_H), dtype=jnp.float32)
    for base, csz in ((0, 32), (32, 32), (64, 32), (96, 8)):
        sl = slice(base, base + csz)
        Dc = D[sl, :]
        phi = jnp.exp(-gamma * (Dc[:, :, None] - mu[None, None, :]) ** 2)
        phif = phi.reshape(csz * P, NUM_BASES)
        h = jnp.maximum(
            jnp.dot(phif, w1_ref[...], preferred_element_type=jnp.float32), 0.0)
        m = jnp.maximum(
            jnp.dot(h, w2_ref[...], preferred_element_type=jnp.float32), 0.0)
        m3 = m.reshape(csz, P, D_H)
        contrib = Xg[sl, None, :] * m3
        acc = acc + jnp.sum(contrib, axis=0)
    out_ref[0] = acc


@jax.jit
def kernel(X, R, batch_index, mu, W1, W2):
    del batch_index
    n = X.shape[0]
    R3 = jnp.pad(R.reshape(N_GRAPHS, P_REAL, R.shape[1]),
                 ((0, 0), (0, P - P_REAL), (0, 8 - R.shape[1])))
    Rt3 = jnp.transpose(R3, (0, 2, 1))
    Xp = jnp.pad(X.reshape(N_GRAPHS, P_REAL, D_H),
                 ((0, 0), (0, P - P_REAL), (0, 0)))
    mu2 = mu.reshape(1, NUM_BASES)

    Hp = pl.pallas_call(
        _cfconv_kernel,
        grid=(N_GRAPHS,),
        in_specs=[
            pl.BlockSpec((1, P, D_H), lambda g: (g, 0, 0)),
            pl.BlockSpec((1, P, 8), lambda g: (g, 0, 0)),
            pl.BlockSpec((1, 8, P), lambda g: (g, 0, 0)),
            pl.BlockSpec((1, NUM_BASES), lambda g: (0, 0)),
            pl.BlockSpec((NUM_BASES, D_H), lambda g: (0, 0)),
            pl.BlockSpec((D_H, D_H), lambda g: (0, 0)),
        ],
        out_specs=pl.BlockSpec((1, P, D_H), lambda g: (g, 0, 0)),
        out_shape=jax.ShapeDtypeStruct((N_GRAPHS, P, D_H), jnp.float32),
        compiler_params=pltpu.CompilerParams(
            dimension_semantics=("parallel",)),
    )(Xp, R3, Rt3, mu2, W1, W2)

    return Hp[:, :P_REAL, :].reshape(n, D_H)


# speedup vs baseline: 1.3537x; 1.1541x over previous
import jax
import jax.numpy as jnp
from jax.experimental import pallas as pl
from jax.experimental.pallas import tpu as pltpu

P = 128
PJ = 104
P_REAL = 100
N_GRAPHS = 100
D_H = 128
NUM_BASES = 64
RADIUS = 4.0
I_CHUNK = 32


def _cfconv_kernel(x_ref, r_ref, rt_ref, mu_ref, w1_ref, w2_ref, out_ref):
    Xg = x_ref[0]
    Rg = r_ref[0]
    Rt = rt_ref[0]
    mu = mu_ref[0]
    delta = mu[1] - mu[0]
    gamma = 1.0 / (2.0 * delta * delta)

    r2c = jnp.sum(Rg * Rg, axis=1)[:, None]
    r2r = jnp.sum(Rt * Rt, axis=0, keepdims=True)
    G = jax.lax.dot_general(Rg, Rg, (((1,), (1,)), ((), ())),
                            preferred_element_type=jnp.float32)
    D_mask = (r2c + r2r) - 2.0 * G
    Dd = ((Rg[:, 0:1] - Rt[0:1, :]) ** 2
          + (Rg[:, 1:2] - Rt[1:2, :]) ** 2
          + (Rg[:, 2:3] - Rt[2:3, :]) ** 2)

    ii = jax.lax.broadcasted_iota(jnp.int32, (P, P), 0)
    jj = jax.lax.broadcasted_iota(jnp.int32, (P, P), 1)
    mask = ((D_mask <= RADIUS * RADIUS) & (ii != jj)
            & (ii < P_REAL) & (jj < P_REAL))
    D = jnp.where(mask, Dd, 1e4)

    acc = jnp.zeros((PJ, D_H), dtype=jnp.float32)
    for base, csz in ((0, 32), (32, 32), (64, 32), (96, 8)):
        sl = slice(base, base + csz)
        Dc = D[sl, :PJ]
        phi = jnp.exp(-gamma * (Dc[:, :, None] - mu[None, None, :]) ** 2)
        phif = phi.reshape(csz * PJ, NUM_BASES)
        h = jnp.maximum(
            jnp.dot(phif, w1_ref[...], preferred_element_type=jnp.float32), 0.0)
        m = jnp.maximum(
            jnp.dot(h, w2_ref[...], preferred_element_type=jnp.float32), 0.0)
        m3 = m.reshape(csz, PJ, D_H)
        contrib = Xg[sl, None, :] * m3
        acc = acc + jnp.sum(contrib, axis=0)
    out_ref[0] = acc


@jax.jit
def kernel(X, R, batch_index, mu, W1, W2):
    del batch_index
    n = X.shape[0]
    R3 = jnp.pad(R.reshape(N_GRAPHS, P_REAL, R.shape[1]),
                 ((0, 0), (0, P - P_REAL), (0, 8 - R.shape[1])))
    Rt3 = jnp.transpose(R3, (0, 2, 1))
    Xp = jnp.pad(X.reshape(N_GRAPHS, P_REAL, D_H),
                 ((0, 0), (0, P - P_REAL), (0, 0)))
    mu2 = mu.reshape(1, NUM_BASES)

    Hp = pl.pallas_call(
        _cfconv_kernel,
        grid=(N_GRAPHS,),
        in_specs=[
            pl.BlockSpec((1, P, D_H), lambda g: (g, 0, 0)),
            pl.BlockSpec((1, P, 8), lambda g: (g, 0, 0)),
            pl.BlockSpec((1, 8, P), lambda g: (g, 0, 0)),
            pl.BlockSpec((1, NUM_BASES), lambda g: (0, 0)),
            pl.BlockSpec((NUM_BASES, D_H), lambda g: (0, 0)),
            pl.BlockSpec((D_H, D_H), lambda g: (0, 0)),
        ],
        out_specs=pl.BlockSpec((1, PJ, D_H), lambda g: (g, 0, 0)),
        out_shape=jax.ShapeDtypeStruct((N_GRAPHS, PJ, D_H), jnp.float32),
        compiler_params=pltpu.CompilerParams(
            dimension_semantics=("parallel",)),
    )(Xp, R3, Rt3, mu2, W1, W2)

    return Hp[:, :P_REAL, :].reshape(n, D_H)


# speedup vs baseline: 1.7349x; 1.2816x over previous
import functools

import jax
import jax.numpy as jnp
from jax.experimental import pallas as pl
from jax.experimental.pallas import tpu as pltpu

P = 128
PJ = 104
P_REAL = 100
N_GRAPHS = 100
D_H = 128
NUM_BASES = 64
RADIUS = 4.0
I_CHUNK = 32


def _cfconv_kernel(x_ref, r_ref, rt_ref, mu_ref, w1_ref, w2_ref, out_ref):
    Xg = x_ref[0]
    Rg = r_ref[0]
    Rt = rt_ref[0]
    mu = mu_ref[0]
    delta = mu[1] - mu[0]
    gamma = 1.0 / (2.0 * delta * delta)

    r2c = jnp.sum(Rg * Rg, axis=1)[:, None]
    r2r = jnp.sum(Rt * Rt, axis=0, keepdims=True)
    G = jax.lax.dot_general(Rg, Rg, (((1,), (1,)), ((), ())),
                            preferred_element_type=jnp.float32)
    D_mask = (r2c + r2r) - 2.0 * G
    Dd = ((Rg[:, 0:1] - Rt[0:1, :]) ** 2
          + (Rg[:, 1:2] - Rt[1:2, :]) ** 2
          + (Rg[:, 2:3] - Rt[2:3, :]) ** 2)

    ii = jax.lax.broadcasted_iota(jnp.int32, (P, P), 0)
    jj = jax.lax.broadcasted_iota(jnp.int32, (P, P), 1)
    mask = ((D_mask <= RADIUS * RADIUS) & (ii != jj)
            & (ii < P_REAL) & (jj < P_REAL))
    D = jnp.where(mask, Dd, 1e4)

    blocks = ((0, 32), (32, 32), (64, 32), (96, 8))
    parts = [[] for _ in blocks]
    for a, (ia, sa) in enumerate(blocks):
        Xa = Xg[ia:ia + sa]
        for b in range(a, len(blocks)):
            jb, sb = blocks[b]
            Dblk = D[ia:ia + sa, jb:jb + sb]
            phi = jnp.exp(-gamma * (Dblk[:, :, None] - mu[None, None, :]) ** 2)
            phif = phi.reshape(sa * sb, NUM_BASES)
            h = jnp.maximum(
                jnp.dot(phif, w1_ref[...],
                        preferred_element_type=jnp.float32), 0.0)
            m = jnp.maximum(
                jnp.dot(h, w2_ref[...],
                        preferred_element_type=jnp.float32), 0.0)
            m3 = m.reshape(sa, sb, D_H)
            parts[b].append(jnp.sum(Xa[:, None, :] * m3, axis=0))
            if b > a:
                Xb = Xg[jb:jb + sb]
                parts[a].append(jnp.sum(Xb[None, :, :] * m3, axis=1))
    out_ref[0] = jnp.concatenate(
        [functools.reduce(lambda u, v: u + v, p) for p in parts], axis=0)


@jax.jit
def kernel(X, R, batch_index, mu, W1, W2):
    del batch_index
    n = X.shape[0]
    R3 = jnp.pad(R.reshape(N_GRAPHS, P_REAL, R.shape[1]),
                 ((0, 0), (0, P - P_REAL), (0, 8 - R.shape[1])))
    Rt3 = jnp.transpose(R3, (0, 2, 1))
    Xp = jnp.pad(X.reshape(N_GRAPHS, P_REAL, D_H),
                 ((0, 0), (0, P - P_REAL), (0, 0)))
    mu2 = mu.reshape(1, NUM_BASES)

    Hp = pl.pallas_call(
        _cfconv_kernel,
        grid=(N_GRAPHS,),
        in_specs=[
            pl.BlockSpec((1, P, D_H), lambda g: (g, 0, 0)),
            pl.BlockSpec((1, P, 8), lambda g: (g, 0, 0)),
            pl.BlockSpec((1, 8, P), lambda g: (g, 0, 0)),
            pl.BlockSpec((1, NUM_BASES), lambda g: (0, 0)),
            pl.BlockSpec((NUM_BASES, D_H), lambda g: (0, 0)),
            pl.BlockSpec((D_H, D_H), lambda g: (0, 0)),
        ],
        out_specs=pl.BlockSpec((1, PJ, D_H), lambda g: (g, 0, 0)),
        out_shape=jax.ShapeDtypeStruct((N_GRAPHS, PJ, D_H), jnp.float32),
        compiler_params=pltpu.CompilerParams(
            dimension_semantics=("parallel",)),
    )(Xp, R3, Rt3, mu2, W1, W2)

    return Hp[:, :P_REAL, :].reshape(n, D_H)


# speedup vs baseline: 1.7974x; 1.0360x over previous
import functools

import jax
import jax.numpy as jnp
from jax.experimental import pallas as pl
from jax.experimental.pallas import tpu as pltpu

P = 128
PJ = 104
P_REAL = 100
N_GRAPHS = 100
G_STEP = 2
D_H = 128
NUM_BASES = 64
RADIUS = 4.0


def _one_graph(Xg, Rg, Rt, mu, w1_ref, w2_ref):
    delta = mu[1] - mu[0]
    gamma = 1.0 / (2.0 * delta * delta)

    r2c = jnp.sum(Rg * Rg, axis=1)[:, None]
    r2r = jnp.sum(Rt * Rt, axis=0, keepdims=True)
    G = jax.lax.dot_general(Rg, Rg, (((1,), (1,)), ((), ())),
                            preferred_element_type=jnp.float32)
    D_mask = (r2c + r2r) - 2.0 * G
    Dd = ((Rg[:, 0:1] - Rt[0:1, :]) ** 2
          + (Rg[:, 1:2] - Rt[1:2, :]) ** 2
          + (Rg[:, 2:3] - Rt[2:3, :]) ** 2)

    ii = jax.lax.broadcasted_iota(jnp.int32, (P, P), 0)
    jj = jax.lax.broadcasted_iota(jnp.int32, (P, P), 1)
    mask = ((D_mask <= RADIUS * RADIUS) & (ii != jj)
            & (ii < P_REAL) & (jj < P_REAL))
    D = jnp.where(mask, Dd, 1e4)

    blocks = ((0, 32), (32, 32), (64, 32), (96, 8))
    parts = [[] for _ in blocks]
    for a, (ia, sa) in enumerate(blocks):
        Xa = Xg[ia:ia + sa]
        for b in range(a, len(blocks)):
            jb, sb = blocks[b]
            Dblk = D[ia:ia + sa, jb:jb + sb]
            phi = jnp.exp(-gamma * (Dblk[:, :, None] - mu[None, None, :]) ** 2)
            phif = phi.reshape(sa * sb, NUM_BASES)
            h = jnp.maximum(
                jnp.dot(phif, w1_ref[...],
                        preferred_element_type=jnp.float32), 0.0)
            m = jnp.maximum(
                jnp.dot(h, w2_ref[...],
                        preferred_element_type=jnp.float32), 0.0)
            m3 = m.reshape(sa, sb, D_H)
            parts[b].append(jnp.sum(Xa[:, None, :] * m3, axis=0))
            if b > a:
                Xb = Xg[jb:jb + sb]
                parts[a].append(jnp.sum(Xb[None, :, :] * m3, axis=1))
    return jnp.concatenate(
        [functools.reduce(lambda u, v: u + v, p) for p in parts], axis=0)


def _cfconv_kernel(x_ref, r_ref, rt_ref, mu_ref, w1_ref, w2_ref, out_ref):
    mu = mu_ref[0]
    for g in range(G_STEP):
        out_ref[g] = _one_graph(x_ref[g], r_ref[g], rt_ref[g], mu,
                                w1_ref, w2_ref)


@jax.jit
def kernel(X, R, batch_index, mu, W1, W2):
    del batch_index
    n = X.shape[0]
    R3 = jnp.pad(R.reshape(N_GRAPHS, P_REAL, R.shape[1]),
                 ((0, 0), (0, P - P_REAL), (0, 8 - R.shape[1])))
    Rt3 = jnp.transpose(R3, (0, 2, 1))
    Xp = jnp.pad(X.reshape(N_GRAPHS, P_REAL, D_H),
                 ((0, 0), (0, P - P_REAL), (0, 0)))
    mu2 = mu.reshape(1, NUM_BASES)

    Hp = pl.pallas_call(
        _cfconv_kernel,
        grid=(N_GRAPHS // G_STEP,),
        in_specs=[
            pl.BlockSpec((G_STEP, P, D_H), lambda g: (g, 0, 0)),
            pl.BlockSpec((G_STEP, P, 8), lambda g: (g, 0, 0)),
            pl.BlockSpec((G_STEP, 8, P), lambda g: (g, 0, 0)),
            pl.BlockSpec((1, NUM_BASES), lambda g: (0, 0)),
            pl.BlockSpec((NUM_BASES, D_H), lambda g: (0, 0)),
            pl.BlockSpec((D_H, D_H), lambda g: (0, 0)),
        ],
        out_specs=pl.BlockSpec((G_STEP, PJ, D_H), lambda g: (g, 0, 0)),
        out_shape=jax.ShapeDtypeStruct((N_GRAPHS, PJ, D_H), jnp.float32),
        compiler_params=pltpu.CompilerParams(
            dimension_semantics=("parallel",)),
    )(Xp, R3, Rt3, mu2, W1, W2)

    return Hp[:, :P_REAL, :].reshape(N_GRAPHS * P_REAL, D_H)[:n]


# speedup vs baseline: 1.8260x; 1.0159x over previous
import functools

import jax
import jax.numpy as jnp
from jax.experimental import pallas as pl
from jax.experimental.pallas import tpu as pltpu

P = 128
PJ = 104
P_REAL = 100
N_GRAPHS = 100
G_STEP = 4
D_H = 128
NUM_BASES = 64
RADIUS = 4.0


def _one_graph(Xg, Rg, Rt, mu, w1_ref, w2_ref):
    delta = mu[1] - mu[0]
    gamma = 1.0 / (2.0 * delta * delta)

    r2c = jnp.sum(Rg * Rg, axis=1)[:, None]
    r2r = jnp.sum(Rt * Rt, axis=0, keepdims=True)
    G = jax.lax.dot_general(Rg, Rg, (((1,), (1,)), ((), ())),
                            preferred_element_type=jnp.float32)
    D_mask = (r2c + r2r) - 2.0 * G
    Dd = ((Rg[:, 0:1] - Rt[0:1, :]) ** 2
          + (Rg[:, 1:2] - Rt[1:2, :]) ** 2
          + (Rg[:, 2:3] - Rt[2:3, :]) ** 2)

    ii = jax.lax.broadcasted_iota(jnp.int32, (P, P), 0)
    jj = jax.lax.broadcasted_iota(jnp.int32, (P, P), 1)
    mask = ((D_mask <= RADIUS * RADIUS) & (ii != jj)
            & (ii < P_REAL) & (jj < P_REAL))
    D = jnp.where(mask, Dd, 1e4)

    blocks = ((0, 32), (32, 32), (64, 32), (96, 8))
    parts = [[] for _ in blocks]
    for a, (ia, sa) in enumerate(blocks):
        Xa = Xg[ia:ia + sa]
        for b in range(a, len(blocks)):
            jb, sb = blocks[b]
            Dblk = D[ia:ia + sa, jb:jb + sb]
            phi = jnp.exp(-gamma * (Dblk[:, :, None] - mu[None, None, :]) ** 2)
            phif = phi.reshape(sa * sb, NUM_BASES)
            h = jnp.maximum(
                jnp.dot(phif, w1_ref[...],
                        preferred_element_type=jnp.float32), 0.0)
            m = jnp.maximum(
                jnp.dot(h, w2_ref[...],
                        preferred_element_type=jnp.float32), 0.0)
            m3 = m.reshape(sa, sb, D_H)
            parts[b].append(jnp.sum(Xa[:, None, :] * m3, axis=0))
            if b > a:
                Xb = Xg[jb:jb + sb]
                parts[a].append(jnp.sum(Xb[None, :, :] * m3, axis=1))
    return jnp.concatenate(
        [functools.reduce(lambda u, v: u + v, p) for p in parts], axis=0)


def _cfconv_kernel(x_ref, r_ref, rt_ref, mu_ref, w1_ref, w2_ref, out_ref):
    mu = mu_ref[0]
    for g in range(G_STEP):
        out_ref[g] = _one_graph(x_ref[g], r_ref[g], rt_ref[g], mu,
                                w1_ref, w2_ref)


@jax.jit
def kernel(X, R, batch_index, mu, W1, W2):
    del batch_index
    n = X.shape[0]
    R3 = jnp.pad(R.reshape(N_GRAPHS, P_REAL, R.shape[1]),
                 ((0, 0), (0, P - P_REAL), (0, 8 - R.shape[1])))
    Rt3 = jnp.transpose(R3, (0, 2, 1))
    Xp = jnp.pad(X.reshape(N_GRAPHS, P_REAL, D_H),
                 ((0, 0), (0, P - P_REAL), (0, 0)))
    mu2 = mu.reshape(1, NUM_BASES)

    Hp = pl.pallas_call(
        _cfconv_kernel,
        grid=(N_GRAPHS // G_STEP,),
        in_specs=[
            pl.BlockSpec((G_STEP, P, D_H), lambda g: (g, 0, 0)),
            pl.BlockSpec((G_STEP, P, 8), lambda g: (g, 0, 0)),
            pl.BlockSpec((G_STEP, 8, P), lambda g: (g, 0, 0)),
            pl.BlockSpec((1, NUM_BASES), lambda g: (0, 0)),
            pl.BlockSpec((NUM_BASES, D_H), lambda g: (0, 0)),
            pl.BlockSpec((D_H, D_H), lambda g: (0, 0)),
        ],
        out_specs=pl.BlockSpec((G_STEP, PJ, D_H), lambda g: (g, 0, 0)),
        out_shape=jax.ShapeDtypeStruct((N_GRAPHS, PJ, D_H), jnp.float32),
        compiler_params=pltpu.CompilerParams(
            dimension_semantics=("parallel",)),
    )(Xp, R3, Rt3, mu2, W1, W2)

    return Hp[:, :P_REAL, :].reshape(N_GRAPHS * P_REAL, D_H)[:n]
